# Initial kernel scaffold; baseline (speedup 1.0000x reference)
import jax
import jax.numpy as jnp
from jax.experimental import pallas as pl


def kernel(x, embeddings):
    raise NotImplementedError("write your pallas kernel here")


# speedup vs baseline: 1.2507x; 1.2507x over previous
import functools

import jax
import jax.numpy as jnp
from jax import lax
from jax.experimental import pallas as pl
from jax.experimental.pallas import tpu as pltpu
from jax.experimental.pallas import tpu_sc as plsc

N_EMB = 8192
DIM = 64
N_ROWS = 16 * 24 * 24
BLK_ROWS = 24 * 24
N_BLK = N_ROWS // BLK_ROWS
BETA = 0.25

SC_CORES = 2
SC_SUBCORES = 16
SC_WORKERS = SC_CORES * SC_SUBCORES
ROWS_PER_WORKER = N_ROWS // SC_WORKERS
GATHER_CHUNK = 96
N_CHUNKS = ROWS_PER_WORKER // GATHER_CHUNK


def _vq_tc_body(x_ref, emb_ref, idx_ref, l_ref):
    xb = x_ref[...]
    emb = emb_ref[...]
    mm = lax.dot_general(
        xb, emb, (((1,), (1,)), ((), ())),
        preferred_element_type=jnp.float32)
    x2 = jnp.sum(xb * xb, axis=1, keepdims=True)
    e2 = jnp.sum(emb * emb, axis=1)
    d = (x2 + e2[None, :]) - 2.0 * mm
    idx_ref[0, 0, :] = jnp.argmin(d, axis=1).astype(jnp.int32)
    dmin = jnp.min(d.reshape(24, 24, N_EMB), axis=2)
    l_ref[0, 0, :] = jnp.sum(dmin, axis=1) * ((1.0 + BETA) / (24.0 * DIM))


@functools.cache
def _make_sc_gather():
    @functools.partial(
        pl.kernel,
        mesh=plsc.VectorSubcoreMesh(core_axis_name="c", subcore_axis_name="s"),
        out_type=jax.ShapeDtypeStruct((N_ROWS, DIM), jnp.float32),
        scratch_types=[
            pltpu.VMEM((GATHER_CHUNK,), jnp.int32),
            pltpu.VMEM((ROWS_PER_WORKER, DIM), jnp.float32),
            pltpu.SemaphoreType.DMA,
        ],
        compiler_params=pltpu.CompilerParams(use_tc_tiling_on_sc=False),
    )
    def _sc_gather(table_hbm, idx_hbm, out_hbm, idx_v, rows_v, sem):
        wid = lax.axis_index("s") * SC_CORES + lax.axis_index("c")
        base = wid * ROWS_PER_WORKER
        for c in range(N_CHUNKS):
            pltpu.sync_copy(
                idx_hbm.at[pl.ds(base + c * GATHER_CHUNK, GATHER_CHUNK)], idx_v)
            pltpu.async_copy(table_hbm.at[idx_v],
                             rows_v.at[pl.ds(c * GATHER_CHUNK, GATHER_CHUNK)],
                             sem).wait()
        pltpu.sync_copy(rows_v, out_hbm.at[pl.ds(base, ROWS_PER_WORKER)])

    return _sc_gather


def kernel(x, embeddings):
    x_flat = x.reshape(N_ROWS, DIM)
    idx3, l3 = pl.pallas_call(
        _vq_tc_body,
        grid=(N_BLK,),
        in_specs=[
            pl.BlockSpec((BLK_ROWS, DIM), lambda i: (i, 0)),
            pl.BlockSpec((N_EMB, DIM), lambda i: (0, 0)),
        ],
        out_specs=[
            pl.BlockSpec((1, 1, BLK_ROWS), lambda i: (i, 0, 0)),
            pl.BlockSpec((1, 1, 24), lambda i: (i, 0, 0)),
        ],
        out_shape=[
            jax.ShapeDtypeStruct((N_BLK, 1, BLK_ROWS), jnp.int32),
            jax.ShapeDtypeStruct((N_BLK, 1, 24), jnp.float32),
        ],
        compiler_params=pltpu.CompilerParams(
            dimension_semantics=("arbitrary",)),
    )(x_flat, embeddings)
    indices = idx3.reshape(N_ROWS)
    x_q = _make_sc_gather()(embeddings, indices)
    return (x_q.reshape(x.shape), l3.reshape(16, 24))


# speedup vs baseline: 1.2892x; 1.0308x over previous
import functools

import jax
import jax.numpy as jnp
from jax import lax
from jax.experimental import pallas as pl
from jax.experimental.pallas import tpu as pltpu
from jax.experimental.pallas import tpu_sc as plsc

N_EMB = 8192
DIM = 64
N_ROWS = 16 * 24 * 24
BLK_ROWS = 24 * 24
N_BLK = N_ROWS // BLK_ROWS
BETA = 0.25

SC_CORES = 2
SC_SUBCORES = 16
SC_WORKERS = SC_CORES * SC_SUBCORES
ROWS_PER_WORKER = N_ROWS // SC_WORKERS
GATHER_CHUNK = 96
N_CHUNKS = ROWS_PER_WORKER // GATHER_CHUNK


def _vq_tc_body(x_ref, emb_ref, idx_ref, l_ref):
    xb = x_ref[...]
    emb = emb_ref[...]
    mm = lax.dot_general(
        xb, emb, (((1,), (1,)), ((), ())),
        preferred_element_type=jnp.float32)
    x2 = jnp.sum(xb * xb, axis=1, keepdims=True)
    e2 = jnp.sum(emb * emb, axis=1) * 0.25
    d = (x2 + e2[None, :]) + mm
    idx_ref[0, 0, :] = jnp.argmin(d, axis=1).astype(jnp.int32)
    dmin = jnp.min(d.reshape(24, 24, N_EMB), axis=2)
    l_ref[0, 0, :] = jnp.sum(dmin, axis=1) * ((1.0 + BETA) / (24.0 * DIM))


@functools.cache
def _make_sc_gather():
    @functools.partial(
        pl.kernel,
        mesh=plsc.VectorSubcoreMesh(core_axis_name="c", subcore_axis_name="s"),
        out_type=jax.ShapeDtypeStruct((N_ROWS, DIM), jnp.float32),
        scratch_types=[
            pltpu.VMEM((N_CHUNKS, GATHER_CHUNK), jnp.int32),
            pltpu.VMEM((ROWS_PER_WORKER, DIM), jnp.float32),
            pltpu.SemaphoreType.DMA,
        ],
        compiler_params=pltpu.CompilerParams(use_tc_tiling_on_sc=False),
    )
    def _sc_gather(table_hbm, idx_hbm, out_hbm, idx_v, rows_v, sem):
        wid = lax.axis_index("s") * SC_CORES + lax.axis_index("c")
        base = wid * ROWS_PER_WORKER
        pltpu.sync_copy(idx_hbm.at[wid], idx_v)
        copies = [
            pltpu.async_copy(
                table_hbm.at[idx_v.at[c]],
                rows_v.at[pl.ds(c * GATHER_CHUNK, GATHER_CHUNK)], sem)
            for c in range(N_CHUNKS)
        ]
        for cp in copies:
            cp.wait()
        pltpu.sync_copy(rows_v, out_hbm.at[pl.ds(base, ROWS_PER_WORKER)])

    return _sc_gather


def kernel(x, embeddings):
    x_flat = x.reshape(N_ROWS, DIM)
    emb_m2 = embeddings * (-2.0)
    idx3, l3 = pl.pallas_call(
        _vq_tc_body,
        grid=(N_BLK,),
        in_specs=[
            pl.BlockSpec((BLK_ROWS, DIM), lambda i: (i, 0)),
            pl.BlockSpec((N_EMB, DIM), lambda i: (0, 0)),
        ],
        out_specs=[
            pl.BlockSpec((1, 1, BLK_ROWS), lambda i: (i, 0, 0)),
            pl.BlockSpec((1, 1, 24), lambda i: (i, 0, 0)),
        ],
        out_shape=[
            jax.ShapeDtypeStruct((N_BLK, 1, BLK_ROWS), jnp.int32),
            jax.ShapeDtypeStruct((N_BLK, 1, 24), jnp.float32),
        ],
        compiler_params=pltpu.CompilerParams(
            dimension_semantics=("arbitrary",)),
    )(x_flat, emb_m2)
    indices = idx3.reshape(SC_WORKERS, N_CHUNKS, GATHER_CHUNK)
    x_q = _make_sc_gather()(embeddings, indices)
    return (x_q.reshape(x.shape), l3.reshape(16, 24))
